# baseline (speedup 1.0000x reference)
import functools

import jax
import jax.numpy as jnp
from jax import lax
from jax.experimental import pallas as pl
from jax.experimental.pallas import tpu as pltpu
from jax.experimental.pallas import tpu_sc as plsc

N = 10000
E = 320000
D_IN = 128
D_OUT = 128
H = 32

NC = 2
NS = 16
NW = NC * NS
CHUNK = 128
NCH = 80
EPW = NCH * CHUNK
EPAD = EPW * NW
NPAD = 10112
RPT = NPAD // NS
DEGW = 16
GP = 8
NHB = 0


def _sc_deg_body(dst_hbm, ones_hbm, zeros_hbm, out_hbm, dstbuf, onesbuf,
                 zbuf, acc):
    c = lax.axis_index("c")
    s = lax.axis_index("s")
    wid = c * NS + s
    pltpu.sync_copy(zeros_hbm, zbuf)
    pltpu.sync_copy(zbuf, acc.at[pl.ds(s * RPT, RPT)])
    pltpu.sync_copy(ones_hbm, onesbuf)
    pltpu.sync_copy(dst_hbm.at[pl.ds(wid * NCH, NCH)], dstbuf)
    plsc.subcore_barrier()

    def chunk(j, carry):
        pltpu.sync_copy(onesbuf, acc.at[dstbuf.at[j]], add=True)
        return carry

    lax.fori_loop(0, NCH, chunk, 0, unroll=False)
    plsc.subcore_barrier()
    pltpu.sync_copy(acc.at[pl.ds(s * RPT, RPT)], zbuf)
    pltpu.sync_copy(zbuf, out_hbm.at[c, pl.ds(s * RPT, RPT)])


def _sc_prop_body(ys_hbm, src_hbm, dst_hbm, zeros_hbm, out_hbm,
                  srcbuf, dstbuf, rows, zbuf, ys_sp, acc,
                  sem0, sem1, sem2, sem3, sem4, sem5, sem6, sem7):
    c = lax.axis_index("c")
    s = lax.axis_index("s")
    wid = c * NS + s
    pltpu.sync_copy(zeros_hbm, zbuf)
    pltpu.sync_copy(zbuf, acc.at[pl.ds(s * RPT, RPT)])
    pltpu.sync_copy(ys_hbm.at[pl.ds(s * RPT, RPT)], zbuf)
    pltpu.sync_copy(zbuf, ys_sp.at[pl.ds(s * RPT, RPT)])
    pltpu.sync_copy(src_hbm.at[pl.ds(wid * NCH, NCH)], srcbuf)
    pltpu.sync_copy(dst_hbm.at[pl.ds(wid * NCH, NCH)], dstbuf)
    plsc.subcore_barrier()

    sems = [sem0, sem1, sem2, sem3, sem4, sem5, sem6, sem7]

    def gather(j, b):
        src_tbl = ys_hbm if b < NHB else ys_sp
        pltpu.async_copy(src_tbl.at[srcbuf.at[j]], rows.at[b], sems[b])

    def gather_wait(j, b):
        src_tbl = ys_hbm if b < NHB else ys_sp
        pltpu.make_async_copy(src_tbl.at[srcbuf.at[j]], rows.at[b],
                              sems[b]).wait()

    for b in range(GP):
        gather(b, b)

    def group(g, carry):
        for b in range(GP):
            j = g * GP + b
            gather_wait(j, b)
            pltpu.sync_copy(rows.at[b], acc.at[dstbuf.at[j]], add=True)

            @pl.when(g + 1 < NCH // GP)
            def _():
                gather(j + GP, b)

        return carry

    lax.fori_loop(0, NCH // GP, group, 0, unroll=False)
    plsc.subcore_barrier()
    pltpu.sync_copy(acc.at[pl.ds(s * RPT, RPT)], zbuf)
    pltpu.sync_copy(zbuf, out_hbm.at[c, pl.ds(s * RPT, RPT)])


@functools.cache
def _sc_kernels():
    mesh = plsc.VectorSubcoreMesh(
        core_axis_name="c", subcore_axis_name="s",
        num_cores=NC, num_subcores=NS)
    params = pltpu.CompilerParams(use_tc_tiling_on_sc=False)
    sc_deg = pl.kernel(
        _sc_deg_body,
        out_type=jax.ShapeDtypeStruct((NC, NPAD, DEGW), jnp.float32),
        mesh=mesh,
        compiler_params=params,
        scratch_types=[
            pltpu.VMEM((NCH, CHUNK), jnp.int32),
            pltpu.VMEM((CHUNK, DEGW), jnp.float32),
            pltpu.VMEM((RPT, DEGW), jnp.float32),
            pltpu.VMEM_SHARED((NPAD, DEGW), jnp.float32),
        ],
    )
    sc_prop = pl.kernel(
        _sc_prop_body,
        out_type=jax.ShapeDtypeStruct((NC, NPAD, H), jnp.float32),
        mesh=mesh,
        compiler_params=params,
        scratch_types=[
            pltpu.VMEM((NCH, CHUNK), jnp.int32),
            pltpu.VMEM((NCH, CHUNK), jnp.int32),
            pltpu.VMEM((GP, CHUNK, H), jnp.float32),
            pltpu.VMEM((RPT, H), jnp.float32),
            pltpu.VMEM_SHARED((NPAD, H), jnp.float32),
            pltpu.VMEM_SHARED((NPAD, H), jnp.float32),
            pltpu.SemaphoreType.DMA,
            pltpu.SemaphoreType.DMA,
            pltpu.SemaphoreType.DMA,
            pltpu.SemaphoreType.DMA,
            pltpu.SemaphoreType.DMA,
            pltpu.SemaphoreType.DMA,
            pltpu.SemaphoreType.DMA,
            pltpu.SemaphoreType.DMA,
        ],
    )
    return sc_deg, sc_prop


def _mynorm(t):
    mn = jnp.min(t, axis=1, keepdims=True)
    mx = jnp.max(t, axis=1, keepdims=True)
    return 2.0 * (t - mn) / (mx - mn + 1e-08) - 1.0


def _tc_head_a_body(x_ref, w_ref, b_ref, x0_ref):
    x0_ref[...] = _mynorm(
        jnp.maximum(x_ref[...] @ w_ref[...] + b_ref[...], 0.0))


def _tc_head_b_body(degp_ref, x0_ref, ys_ref, dis_ref):
    deg = degp_ref[0, :, 0:1] + degp_ref[1, :, 0:1] + 1.0
    dis = lax.rsqrt(deg)
    dis_ref[...] = dis
    ys_ref[...] = x0_ref[...] * dis


def _tc_layer_body(p_ref, xprev_ref, dis_ref, w_ref, b_ref, xk_ref, ys_ref):
    dis = dis_ref[...]
    ssum = p_ref[0] + p_ref[1]
    agg = dis * ssum + (dis * dis) * xprev_ref[...]
    xk = agg @ w_ref[...] + b_ref[...]
    xk_ref[...] = xk
    ys_ref[...] = xk * dis


def _tc_tail_a_body(x0_ref, x1_ref, x2_ref, x3_ref, w5_ref, b5_ref,
                    part_ref):
    x0 = x0_ref[...]
    x1 = x1_ref[...]
    m0 = _mynorm(x0)
    m1 = _mynorm(x1)
    m2 = _mynorm(x2_ref[...])
    m3 = _mynorm(x3_ref[...])
    w5 = w5_ref[...]
    part_ref[...] = (x0 @ w5[0:H, :]
                     + x1 @ w5[H:2 * H, :]
                     + (m2 - m0) @ w5[2 * H:3 * H, :]
                     + (m3 - m1) @ w5[3 * H:4 * H, :]
                     - m2 @ w5[4 * H:5 * H, :]
                     + b5_ref[...])


def _tc_tail_b_body(p_ref, x3_ref, dis_ref, w4_ref, b4_ref, part_ref,
                    w5_ref, out_ref):
    dis = dis_ref[...]
    ssum = p_ref[0] + p_ref[1]
    agg = dis * ssum + (dis * dis) * x3_ref[...]
    x4 = agg @ w4_ref[...] + b4_ref[...]
    out_ref[...] = part_ref[...] + _mynorm(x4) @ w5_ref[4 * H:5 * H, :]


_f32 = jnp.float32
BN = NPAD
G = NPAD // BN


def _rows(c):
    return pl.BlockSpec((BN, c), lambda i: (i, 0))


def _prows(c):
    return pl.BlockSpec((NC, BN, c), lambda i: (0, i, 0))


def _full(r, c):
    return pl.BlockSpec((r, c), lambda i: (0, 0))


_tc_head_a = pl.pallas_call(
    _tc_head_a_body,
    grid=(G,),
    in_specs=[_rows(D_IN), _full(D_IN, H), _full(1, H)],
    out_specs=_rows(H),
    out_shape=jax.ShapeDtypeStruct((NPAD, H), _f32),
)

_tc_head_b = pl.pallas_call(
    _tc_head_b_body,
    grid=(G,),
    in_specs=[_prows(DEGW), _rows(H)],
    out_specs=(_rows(H), _rows(1)),
    out_shape=(
        jax.ShapeDtypeStruct((NPAD, H), _f32),
        jax.ShapeDtypeStruct((NPAD, 1), _f32),
    ),
)

_tc_layer = pl.pallas_call(
    _tc_layer_body,
    grid=(G,),
    in_specs=[_prows(H), _rows(H), _rows(1), _full(H, H), _full(1, H)],
    out_specs=(_rows(H), _rows(H)),
    out_shape=(
        jax.ShapeDtypeStruct((NPAD, H), _f32),
        jax.ShapeDtypeStruct((NPAD, H), _f32),
    ),
)

_tc_tail_a = pl.pallas_call(
    _tc_tail_a_body,
    grid=(G,),
    in_specs=[_rows(H), _rows(H), _rows(H), _rows(H),
              _full(5 * H, D_OUT), _full(1, D_OUT)],
    out_specs=_rows(D_OUT),
    out_shape=jax.ShapeDtypeStruct((NPAD, D_OUT), _f32),
)

_tc_tail_b = pl.pallas_call(
    _tc_tail_b_body,
    grid=(G,),
    in_specs=[_prows(H), _rows(H), _rows(1), _full(H, H), _full(1, H),
              _rows(D_OUT), _full(5 * H, D_OUT)],
    out_specs=_rows(D_OUT),
    out_shape=jax.ShapeDtypeStruct((NPAD, D_OUT), _f32),
)


def kernel(x, edge_index, fc1_W, fc1_b, W1, b1, W2, b2, W3, b3, W4, b4,
           W5, b5):
    src = edge_index[0]
    dst = edge_index[1]
    pad = EPAD - E
    padv = jnp.full((pad,), N, jnp.int32)
    src2 = jnp.concatenate([src, padv]).reshape(NW * NCH, CHUNK)
    dst2 = jnp.concatenate([dst, padv]).reshape(NW * NCH, CHUNK)
    xp = jnp.pad(x, ((0, NPAD - N), (0, 0)))

    ones_deg = jnp.ones((CHUNK, DEGW), _f32)
    zeros_deg = jnp.zeros((RPT, DEGW), _f32)
    zeros_h = jnp.zeros((RPT, H), _f32)

    _sc_deg, _sc_prop = _sc_kernels()
    degp = _sc_deg(dst2, ones_deg, zeros_deg)

    x0 = _tc_head_a(xp, fc1_W, fc1_b.reshape(1, H))
    ys0, dis = _tc_head_b(degp, x0)

    p1 = _sc_prop(ys0, src2, dst2, zeros_h)
    x1, ys1 = _tc_layer(p1, x0, dis, W1, b1.reshape(1, H))

    p2 = _sc_prop(ys1, src2, dst2, zeros_h)
    x2, ys2 = _tc_layer(p2, x1, dis, W2, b2.reshape(1, H))

    p3 = _sc_prop(ys2, src2, dst2, zeros_h)
    x3, ys3 = _tc_layer(p3, x2, dis, W3, b3.reshape(1, H))

    p4 = _sc_prop(ys3, src2, dst2, zeros_h)
    part = _tc_tail_a(x0, x1, x2, x3, W5, b5.reshape(1, D_OUT))
    x5 = _tc_tail_b(p4, x3, dis, W4, b4.reshape(1, H), part, W5)
    return x5[:N]


# speedup vs baseline: 1.0300x; 1.0300x over previous
import functools

import jax
import jax.numpy as jnp
from jax import lax
from jax.experimental import pallas as pl
from jax.experimental.pallas import tpu as pltpu
from jax.experimental.pallas import tpu_sc as plsc

N = 10000
E = 320000
D_IN = 128
D_OUT = 128
H = 32

NC = 2
NS = 16
NW = NC * NS
CHUNK = 128
NCH = 80
EPW = NCH * CHUNK
EPAD = EPW * NW
NPAD = 10112
RPT = NPAD // NS
DEGW = 16
GP = 8
NHB = 0


def _sc_deg_body(dst_hbm, ones_hbm, zeros_hbm, out_hbm, dstbuf, onesbuf,
                 zbuf, acc):
    c = lax.axis_index("c")
    s = lax.axis_index("s")
    wid = c * NS + s
    pltpu.sync_copy(zeros_hbm, zbuf)
    pltpu.sync_copy(zbuf, acc.at[pl.ds(s * RPT, RPT)])
    pltpu.sync_copy(ones_hbm, onesbuf)
    pltpu.sync_copy(dst_hbm.at[pl.ds(wid * NCH, NCH)], dstbuf)
    plsc.subcore_barrier()

    def chunk(j, carry):
        pltpu.sync_copy(onesbuf, acc.at[dstbuf.at[j]], add=True)
        return carry

    lax.fori_loop(0, NCH, chunk, 0, unroll=False)
    plsc.subcore_barrier()
    pltpu.sync_copy(acc.at[pl.ds(s * RPT, RPT)], zbuf)
    pltpu.sync_copy(zbuf, out_hbm.at[c, pl.ds(s * RPT, RPT)])


def _sc_prop_body(ys_hbm, src_hbm, dst_hbm, zeros_hbm, out_hbm,
                  srcbuf, dstbuf, rows, zbuf, ys_sp, acc,
                  sem0, sem1, sem2, sem3, sem4, sem5, sem6, sem7):
    c = lax.axis_index("c")
    s = lax.axis_index("s")
    wid = c * NS + s
    pltpu.sync_copy(zeros_hbm, zbuf)
    pltpu.sync_copy(zbuf, acc.at[pl.ds(s * RPT, RPT)])
    pltpu.sync_copy(ys_hbm.at[pl.ds(s * RPT, RPT)], zbuf)
    pltpu.sync_copy(zbuf, ys_sp.at[pl.ds(s * RPT, RPT)])
    pltpu.sync_copy(src_hbm.at[pl.ds(wid * NCH, NCH)], srcbuf)
    pltpu.sync_copy(dst_hbm.at[pl.ds(wid * NCH, NCH)], dstbuf)
    plsc.subcore_barrier()

    sems = [sem0, sem1, sem2, sem3, sem4, sem5, sem6, sem7]

    def gather(j, b):
        src_tbl = ys_hbm if b < NHB else ys_sp
        pltpu.async_copy(src_tbl.at[srcbuf.at[j]], rows.at[b], sems[b])

    def gather_wait(j, b):
        src_tbl = ys_hbm if b < NHB else ys_sp
        pltpu.make_async_copy(src_tbl.at[srcbuf.at[j]], rows.at[b],
                              sems[b]).wait()

    for b in range(GP):
        gather(b, b)

    def group(g, carry):
        for b in range(GP):
            j = g * GP + b
            gather_wait(j, b)
            pltpu.sync_copy(rows.at[b], acc.at[dstbuf.at[j]], add=True)

            @pl.when(g + 1 < NCH // GP)
            def _():
                gather(j + GP, b)

        return carry

    lax.fori_loop(0, NCH // GP, group, 0, unroll=False)
    plsc.subcore_barrier()
    pltpu.sync_copy(acc.at[pl.ds(s * RPT, RPT)], zbuf)
    pltpu.sync_copy(zbuf, out_hbm.at[c, pl.ds(s * RPT, RPT)])


@functools.cache
def _sc_kernels():
    mesh = plsc.VectorSubcoreMesh(
        core_axis_name="c", subcore_axis_name="s",
        num_cores=NC, num_subcores=NS)
    params = pltpu.CompilerParams(use_tc_tiling_on_sc=False)
    sc_deg = pl.kernel(
        _sc_deg_body,
        out_type=jax.ShapeDtypeStruct((NC, NPAD, DEGW), jnp.float32),
        mesh=mesh,
        compiler_params=params,
        scratch_types=[
            pltpu.VMEM((NCH, CHUNK), jnp.int32),
            pltpu.VMEM((CHUNK, DEGW), jnp.float32),
            pltpu.VMEM((RPT, DEGW), jnp.float32),
            pltpu.VMEM_SHARED((NPAD, DEGW), jnp.float32),
        ],
    )
    sc_prop = pl.kernel(
        _sc_prop_body,
        out_type=jax.ShapeDtypeStruct((NC, NPAD, H), jnp.float32),
        mesh=mesh,
        compiler_params=params,
        scratch_types=[
            pltpu.VMEM((NCH, CHUNK), jnp.int32),
            pltpu.VMEM((NCH, CHUNK), jnp.int32),
            pltpu.VMEM((GP, CHUNK, H), jnp.float32),
            pltpu.VMEM((RPT, H), jnp.float32),
            pltpu.VMEM_SHARED((NPAD, H), jnp.float32),
            pltpu.VMEM_SHARED((NPAD, H), jnp.float32),
            pltpu.SemaphoreType.DMA,
            pltpu.SemaphoreType.DMA,
            pltpu.SemaphoreType.DMA,
            pltpu.SemaphoreType.DMA,
            pltpu.SemaphoreType.DMA,
            pltpu.SemaphoreType.DMA,
            pltpu.SemaphoreType.DMA,
            pltpu.SemaphoreType.DMA,
        ],
    )
    return sc_deg, sc_prop


def _mynorm(t):
    mn = jnp.min(t, axis=1, keepdims=True)
    mx = jnp.max(t, axis=1, keepdims=True)
    return 2.0 * (t - mn) / (mx - mn + 1e-08) - 1.0


def _tc_head_a_body(x_ref, w_ref, b_ref, x0_ref):
    x0_ref[...] = _mynorm(
        jnp.maximum(x_ref[...] @ w_ref[...] + b_ref[...], 0.0))


def _tc_head_b_body(degp_ref, x0_ref, ys_ref, dis_ref):
    deg = degp_ref[0, :, 0:1] + degp_ref[1, :, 0:1] + 1.0
    dis = lax.rsqrt(deg)
    dis_ref[...] = dis
    ys_ref[...] = x0_ref[...] * dis


def _tc_layer_body(p_ref, xprev_ref, dis_ref, w_ref, b_ref, xk_ref, ys_ref):
    dis = dis_ref[...]
    ssum = p_ref[0] + p_ref[1]
    agg = dis * ssum + (dis * dis) * xprev_ref[...]
    xk = agg @ w_ref[...] + b_ref[...]
    xk_ref[...] = xk
    ys_ref[...] = xk * dis


def _tc_tail_a_body(x0_ref, x1_ref, x2_ref, x3_ref, w5_ref, b5_ref,
                    part_ref):
    x0 = x0_ref[...]
    x1 = x1_ref[...]
    m0 = _mynorm(x0)
    m1 = _mynorm(x1)
    m2 = _mynorm(x2_ref[...])
    m3 = _mynorm(x3_ref[...])
    w5 = w5_ref[...]
    part_ref[...] = (x0 @ w5[0:H, :]
                     + x1 @ w5[H:2 * H, :]
                     + (m2 - m0) @ w5[2 * H:3 * H, :]
                     + (m3 - m1) @ w5[3 * H:4 * H, :]
                     - m2 @ w5[4 * H:5 * H, :]
                     + b5_ref[...])


def _tc_tail_b_body(p_ref, x3_ref, dis_ref, w4_ref, b4_ref, part_ref,
                    w5_ref, out_ref):
    dis = dis_ref[...]
    ssum = p_ref[0] + p_ref[1]
    agg = dis * ssum + (dis * dis) * x3_ref[...]
    x4 = agg @ w4_ref[...] + b4_ref[...]
    out_ref[...] = part_ref[...] + _mynorm(x4) @ w5_ref[4 * H:5 * H, :]


_f32 = jnp.float32
BN = 2528
G = NPAD // BN


def _rows(c):
    return pl.BlockSpec((BN, c), lambda i: (i, 0))


def _prows(c):
    return pl.BlockSpec((NC, BN, c), lambda i: (0, i, 0))


def _full(r, c):
    return pl.BlockSpec((r, c), lambda i: (0, 0))


_tc_head_a = pl.pallas_call(
    _tc_head_a_body,
    grid=(G,),
    in_specs=[_rows(D_IN), _full(D_IN, H), _full(1, H)],
    out_specs=_rows(H),
    out_shape=jax.ShapeDtypeStruct((NPAD, H), _f32),
)

_tc_head_b = pl.pallas_call(
    _tc_head_b_body,
    grid=(G,),
    in_specs=[_prows(DEGW), _rows(H)],
    out_specs=(_rows(H), _rows(1)),
    out_shape=(
        jax.ShapeDtypeStruct((NPAD, H), _f32),
        jax.ShapeDtypeStruct((NPAD, 1), _f32),
    ),
)

_tc_layer = pl.pallas_call(
    _tc_layer_body,
    grid=(G,),
    in_specs=[_prows(H), _rows(H), _rows(1), _full(H, H), _full(1, H)],
    out_specs=(_rows(H), _rows(H)),
    out_shape=(
        jax.ShapeDtypeStruct((NPAD, H), _f32),
        jax.ShapeDtypeStruct((NPAD, H), _f32),
    ),
)

_tc_tail_a = pl.pallas_call(
    _tc_tail_a_body,
    grid=(G,),
    in_specs=[_rows(H), _rows(H), _rows(H), _rows(H),
              _full(5 * H, D_OUT), _full(1, D_OUT)],
    out_specs=_rows(D_OUT),
    out_shape=jax.ShapeDtypeStruct((NPAD, D_OUT), _f32),
)

_tc_tail_b = pl.pallas_call(
    _tc_tail_b_body,
    grid=(G,),
    in_specs=[_prows(H), _rows(H), _rows(1), _full(H, H), _full(1, H),
              _rows(D_OUT), _full(5 * H, D_OUT)],
    out_specs=_rows(D_OUT),
    out_shape=jax.ShapeDtypeStruct((NPAD, D_OUT), _f32),
)


def kernel(x, edge_index, fc1_W, fc1_b, W1, b1, W2, b2, W3, b3, W4, b4,
           W5, b5):
    src = edge_index[0]
    dst = edge_index[1]
    pad = EPAD - E
    padv = jnp.full((pad,), N, jnp.int32)
    src2 = jnp.concatenate([src, padv]).reshape(NW * NCH, CHUNK)
    dst2 = jnp.concatenate([dst, padv]).reshape(NW * NCH, CHUNK)
    xp = jnp.pad(x, ((0, NPAD - N), (0, 0)))

    ones_deg = jnp.ones((CHUNK, DEGW), _f32)
    zeros_deg = jnp.zeros((RPT, DEGW), _f32)
    zeros_h = jnp.zeros((RPT, H), _f32)

    _sc_deg, _sc_prop = _sc_kernels()
    degp = _sc_deg(dst2, ones_deg, zeros_deg)

    x0 = _tc_head_a(xp, fc1_W, fc1_b.reshape(1, H))
    ys0, dis = _tc_head_b(degp, x0)

    p1 = _sc_prop(ys0, src2, dst2, zeros_h)
    x1, ys1 = _tc_layer(p1, x0, dis, W1, b1.reshape(1, H))

    p2 = _sc_prop(ys1, src2, dst2, zeros_h)
    x2, ys2 = _tc_layer(p2, x1, dis, W2, b2.reshape(1, H))

    p3 = _sc_prop(ys2, src2, dst2, zeros_h)
    x3, ys3 = _tc_layer(p3, x2, dis, W3, b3.reshape(1, H))

    p4 = _sc_prop(ys3, src2, dst2, zeros_h)
    part = _tc_tail_a(x0, x1, x2, x3, W5, b5.reshape(1, D_OUT))
    x5 = _tc_tail_b(p4, x3, dis, W4, b4.reshape(1, H), part, W5)
    return x5[:N]


# speedup vs baseline: 1.0648x; 1.0337x over previous
import functools

import jax
import jax.numpy as jnp
from jax import lax
from jax.experimental import pallas as pl
from jax.experimental.pallas import tpu as pltpu
from jax.experimental.pallas import tpu_sc as plsc

N = 10000
E = 320000
D_IN = 128
D_OUT = 128
H = 32

NC = 2
NS = 16
NW = NC * NS
CHUNK = 128
NCH = 80
EPW = NCH * CHUNK
EPAD = EPW * NW
NPAD = 10112
RPT = NPAD // NS
DEGW = 16
RING = 10
PD = 5


def _sc_deg_body(dst_hbm, ones_hbm, zeros_hbm, out_hbm, dstbuf, onesbuf,
                 zbuf, acc):
    c = lax.axis_index("c")
    s = lax.axis_index("s")
    wid = c * NS + s
    pltpu.sync_copy(zeros_hbm, zbuf)
    pltpu.sync_copy(zbuf, acc.at[pl.ds(s * RPT, RPT)])
    pltpu.sync_copy(ones_hbm, onesbuf)
    pltpu.sync_copy(dst_hbm.at[pl.ds(wid * NCH, NCH)], dstbuf)
    plsc.subcore_barrier()

    def chunk(j, carry):
        pltpu.sync_copy(onesbuf, acc.at[dstbuf.at[j]], add=True)
        return carry

    lax.fori_loop(0, NCH, chunk, 0, unroll=False)
    plsc.subcore_barrier()
    pltpu.sync_copy(acc.at[pl.ds(s * RPT, RPT)], zbuf)
    pltpu.sync_copy(zbuf, out_hbm.at[c, pl.ds(s * RPT, RPT)])


def _sc_prop_body(ys_hbm, src_hbm, dst_hbm, zeros_hbm, out_hbm,
                  srcbuf, dstbuf, rows, zbuf, ys_sp, acc,
                  *sems):
    c = lax.axis_index("c")
    s = lax.axis_index("s")
    wid = c * NS + s
    pltpu.sync_copy(zeros_hbm, zbuf)
    pltpu.sync_copy(zbuf, acc.at[pl.ds(s * RPT, RPT)])
    pltpu.sync_copy(ys_hbm.at[pl.ds(s * RPT, RPT)], zbuf)
    pltpu.sync_copy(zbuf, ys_sp.at[pl.ds(s * RPT, RPT)])
    pltpu.sync_copy(src_hbm.at[pl.ds(wid * NCH, NCH)], srcbuf)
    pltpu.sync_copy(dst_hbm.at[pl.ds(wid * NCH, NCH)], dstbuf)
    plsc.subcore_barrier()

    gsems = sems[:RING]
    ssems = sems[RING:]

    def gather(j, b):
        pltpu.async_copy(ys_sp.at[srcbuf.at[j]], rows.at[b], gsems[b])

    def gather_wait(j, b):
        pltpu.make_async_copy(ys_sp.at[srcbuf.at[j]], rows.at[b],
                              gsems[b]).wait()

    def scatter(j, b):
        pltpu.async_copy(rows.at[b], acc.at[dstbuf.at[j]], ssems[b],
                         add=True)

    def scatter_wait(j, b):
        pltpu.make_async_copy(rows.at[b], acc.at[dstbuf.at[j]],
                              ssems[b]).wait()

    for b in range(PD):
        gather(b, b)

    def group(g, carry):
        for b in range(RING):
            j = g * RING + b
            gather_wait(j, b)
            scatter(j, b)
            jp = j + PD
            bp = (b + PD) % RING

            @pl.when(jp < NCH)
            def _():
                @pl.when(jp >= RING)
                def _():
                    scatter_wait(jp - RING, bp)

                gather(jp, bp)

        return carry

    lax.fori_loop(0, NCH // RING, group, 0, unroll=False)
    for b in range(RING):
        scatter_wait(NCH - RING + b, b)
    plsc.subcore_barrier()
    pltpu.sync_copy(acc.at[pl.ds(s * RPT, RPT)], zbuf)
    pltpu.sync_copy(zbuf, out_hbm.at[c, pl.ds(s * RPT, RPT)])


@functools.cache
def _sc_kernels():
    mesh = plsc.VectorSubcoreMesh(
        core_axis_name="c", subcore_axis_name="s",
        num_cores=NC, num_subcores=NS)
    params = pltpu.CompilerParams(use_tc_tiling_on_sc=False)
    sc_deg = pl.kernel(
        _sc_deg_body,
        out_type=jax.ShapeDtypeStruct((NC, NPAD, DEGW), jnp.float32),
        mesh=mesh,
        compiler_params=params,
        scratch_types=[
            pltpu.VMEM((NCH, CHUNK), jnp.int32),
            pltpu.VMEM((CHUNK, DEGW), jnp.float32),
            pltpu.VMEM((RPT, DEGW), jnp.float32),
            pltpu.VMEM_SHARED((NPAD, DEGW), jnp.float32),
        ],
    )
    sc_prop = pl.kernel(
        _sc_prop_body,
        out_type=jax.ShapeDtypeStruct((NC, NPAD, H), jnp.float32),
        mesh=mesh,
        compiler_params=params,
        scratch_types=[
            pltpu.VMEM((NCH, CHUNK), jnp.int32),
            pltpu.VMEM((NCH, CHUNK), jnp.int32),
            pltpu.VMEM((RING, CHUNK, H), jnp.float32),
            pltpu.VMEM((RPT, H), jnp.float32),
            pltpu.VMEM_SHARED((NPAD, H), jnp.float32),
            pltpu.VMEM_SHARED((NPAD, H), jnp.float32),
        ] + [pltpu.SemaphoreType.DMA] * (2 * RING),
    )
    return sc_deg, sc_prop


def _mynorm(t):
    mn = jnp.min(t, axis=1, keepdims=True)
    mx = jnp.max(t, axis=1, keepdims=True)
    return 2.0 * (t - mn) / (mx - mn + 1e-08) - 1.0


def _tc_head_a_body(x_ref, w_ref, b_ref, x0_ref):
    x0_ref[...] = _mynorm(
        jnp.maximum(x_ref[...] @ w_ref[...] + b_ref[...], 0.0))


def _tc_head_b_body(degp_ref, x0_ref, ys_ref, dis_ref):
    deg = degp_ref[0, :, 0:1] + degp_ref[1, :, 0:1] + 1.0
    dis = lax.rsqrt(deg)
    dis_ref[...] = dis
    ys_ref[...] = x0_ref[...] * dis


def _tc_layer_body(p_ref, xprev_ref, dis_ref, w_ref, b_ref, xk_ref, ys_ref):
    dis = dis_ref[...]
    ssum = p_ref[0] + p_ref[1]
    agg = dis * ssum + (dis * dis) * xprev_ref[...]
    xk = agg @ w_ref[...] + b_ref[...]
    xk_ref[...] = xk
    ys_ref[...] = xk * dis


def _tc_tail_a_body(x0_ref, x1_ref, x2_ref, x3_ref, w5_ref, b5_ref,
                    part_ref):
    x0 = x0_ref[...]
    x1 = x1_ref[...]
    m0 = _mynorm(x0)
    m1 = _mynorm(x1)
    m2 = _mynorm(x2_ref[...])
    m3 = _mynorm(x3_ref[...])
    w5 = w5_ref[...]
    part_ref[...] = (x0 @ w5[0:H, :]
                     + x1 @ w5[H:2 * H, :]
                     + (m2 - m0) @ w5[2 * H:3 * H, :]
                     + (m3 - m1) @ w5[3 * H:4 * H, :]
                     - m2 @ w5[4 * H:5 * H, :]
                     + b5_ref[...])


def _tc_tail_b_body(p_ref, x3_ref, dis_ref, w4_ref, b4_ref, part_ref,
                    w5_ref, out_ref):
    dis = dis_ref[...]
    ssum = p_ref[0] + p_ref[1]
    agg = dis * ssum + (dis * dis) * x3_ref[...]
    x4 = agg @ w4_ref[...] + b4_ref[...]
    out_ref[...] = part_ref[...] + _mynorm(x4) @ w5_ref[4 * H:5 * H, :]


_f32 = jnp.float32
BN = 2528
G = NPAD // BN


def _rows(c):
    return pl.BlockSpec((BN, c), lambda i: (i, 0))


def _prows(c):
    return pl.BlockSpec((NC, BN, c), lambda i: (0, i, 0))


def _full(r, c):
    return pl.BlockSpec((r, c), lambda i: (0, 0))


_tc_head_a = pl.pallas_call(
    _tc_head_a_body,
    grid=(G,),
    in_specs=[_rows(D_IN), _full(D_IN, H), _full(1, H)],
    out_specs=_rows(H),
    out_shape=jax.ShapeDtypeStruct((NPAD, H), _f32),
)

_tc_head_b = pl.pallas_call(
    _tc_head_b_body,
    grid=(G,),
    in_specs=[_prows(DEGW), _rows(H)],
    out_specs=(_rows(H), _rows(1)),
    out_shape=(
        jax.ShapeDtypeStruct((NPAD, H), _f32),
        jax.ShapeDtypeStruct((NPAD, 1), _f32),
    ),
)

_tc_layer = pl.pallas_call(
    _tc_layer_body,
    grid=(G,),
    in_specs=[_prows(H), _rows(H), _rows(1), _full(H, H), _full(1, H)],
    out_specs=(_rows(H), _rows(H)),
    out_shape=(
        jax.ShapeDtypeStruct((NPAD, H), _f32),
        jax.ShapeDtypeStruct((NPAD, H), _f32),
    ),
)

_tc_tail_a = pl.pallas_call(
    _tc_tail_a_body,
    grid=(G,),
    in_specs=[_rows(H), _rows(H), _rows(H), _rows(H),
              _full(5 * H, D_OUT), _full(1, D_OUT)],
    out_specs=_rows(D_OUT),
    out_shape=jax.ShapeDtypeStruct((NPAD, D_OUT), _f32),
)

_tc_tail_b = pl.pallas_call(
    _tc_tail_b_body,
    grid=(G,),
    in_specs=[_prows(H), _rows(H), _rows(1), _full(H, H), _full(1, H),
              _rows(D_OUT), _full(5 * H, D_OUT)],
    out_specs=_rows(D_OUT),
    out_shape=jax.ShapeDtypeStruct((NPAD, D_OUT), _f32),
)


def kernel(x, edge_index, fc1_W, fc1_b, W1, b1, W2, b2, W3, b3, W4, b4,
           W5, b5):
    src = edge_index[0]
    dst = edge_index[1]
    pad = EPAD - E
    padv = jnp.full((pad,), N, jnp.int32)
    src2 = jnp.concatenate([src, padv]).reshape(NW * NCH, CHUNK)
    dst2 = jnp.concatenate([dst, padv]).reshape(NW * NCH, CHUNK)
    xp = jnp.pad(x, ((0, NPAD - N), (0, 0)))

    ones_deg = jnp.ones((CHUNK, DEGW), _f32)
    zeros_deg = jnp.zeros((RPT, DEGW), _f32)
    zeros_h = jnp.zeros((RPT, H), _f32)

    _sc_deg, _sc_prop = _sc_kernels()
    degp = _sc_deg(dst2, ones_deg, zeros_deg)

    x0 = _tc_head_a(xp, fc1_W, fc1_b.reshape(1, H))
    ys0, dis = _tc_head_b(degp, x0)

    p1 = _sc_prop(ys0, src2, dst2, zeros_h)
    x1, ys1 = _tc_layer(p1, x0, dis, W1, b1.reshape(1, H))

    p2 = _sc_prop(ys1, src2, dst2, zeros_h)
    x2, ys2 = _tc_layer(p2, x1, dis, W2, b2.reshape(1, H))

    p3 = _sc_prop(ys2, src2, dst2, zeros_h)
    x3, ys3 = _tc_layer(p3, x2, dis, W3, b3.reshape(1, H))

    p4 = _sc_prop(ys3, src2, dst2, zeros_h)
    part = _tc_tail_a(x0, x1, x2, x3, W5, b5.reshape(1, D_OUT))
    x5 = _tc_tail_b(p4, x3, dis, W4, b4.reshape(1, H), part, W5)
    return x5[:N]
